# Initial kernel scaffold; baseline (speedup 1.0000x reference)
import jax
import jax.numpy as jnp
from jax.experimental import pallas as pl


def kernel(h, edge_index, W1, attn_l1, attn_r1, b1, W2, attn_l2, attn_r2, b2):
    raise NotImplementedError("write your pallas kernel here")


# speedup vs baseline: 14.0132x; 14.0132x over previous
import functools

import jax
import jax.numpy as jnp
from jax import lax
from jax.experimental import pallas as pl
from jax.experimental.pallas import tpu as pltpu
from jax.experimental.pallas import tpu_sc as plsc

N = 10000
E = 320000
IN_DIM = 128
HID = 64
HEADS = 8
OUT = 128

NC = 2
NS = 16
NW = NC * NS
LANES = 16
EW = E // NW
K = 80
NCH = EW // K
RPT = N // NS

BN = 1000
GRID = N // BN

_mesh = plsc.VectorSubcoreMesh(
    core_axis_name="c", subcore_axis_name="s", num_cores=NC, num_subcores=NS)


def _make_edge_scores(H):
  def body(src_r, dst_r, el_r, er_r, m_r, ex_r, sp_r,
           el_v, er_v, s_v, src_v, dst_v, exb, m_v):
    c = lax.axis_index("c")
    s = lax.axis_index("s")
    w = s * NC + c
    pltpu.sync_copy(el_r, el_v)
    pltpu.sync_copy(er_r, er_v)
    pltpu.sync_copy(m_r, m_v)
    zeros16 = jnp.zeros((LANES,), jnp.float32)

    def zb(i, carry):
      s_v[pl.ds(i * LANES, LANES)] = zeros16
      return carry
    lax.fori_loop(0, (N * H) // LANES, zb, 0)

    mh = [m_v[h, pl.ds(0, LANES)] for h in range(H)]

    def chunk(i, carry):
      base = w * EW + i * K
      pltpu.sync_copy(src_r.at[pl.ds(base, K)], src_v)
      pltpu.sync_copy(dst_r.at[pl.ds(base, K)], dst_v)
      for g in range(K // LANES):
        sv = src_v[pl.ds(g * LANES, LANES)]
        dv = dst_v[pl.ds(g * LANES, LANES)]
        for h in range(H):
          il = plsc.load_gather(el_v, [sv * H + h])
          ir = plsc.load_gather(er_v, [dv * H + h])
          x = il + ir
          x = jnp.maximum(x, 0.2 * x)
          exv = jnp.exp(x - mh[h])
          plsc.addupdate_scatter(s_v, [dv * H + h], exv)
          exb[h, pl.ds(g * LANES, LANES)] = exv
      for h in range(H):
        pltpu.sync_copy(exb.at[h], ex_r.at[pl.ds(h * E + base, K)])
      return carry
    lax.fori_loop(0, NCH, chunk, 0)
    pltpu.sync_copy(s_v, sp_r.at[pl.ds(w * N * H, N * H)])

  return pl.kernel(
      body,
      out_type=(jax.ShapeDtypeStruct((H * E,), jnp.float32),
                jax.ShapeDtypeStruct((NW * N * H,), jnp.float32)),
      mesh=_mesh,
      compiler_params=pltpu.CompilerParams(needs_layout_passes=False),
      scratch_types=[
          pltpu.VMEM((N * H,), jnp.float32),
          pltpu.VMEM((N * H,), jnp.float32),
          pltpu.VMEM((N * H,), jnp.float32),
          pltpu.VMEM((K,), jnp.int32),
          pltpu.VMEM((K,), jnp.int32),
          pltpu.VMEM((H, K), jnp.float32),
          pltpu.VMEM((H, LANES), jnp.float32),
      ],
  )


def _make_aggregate(H2):
  nq = 128 // (LANES * H2)

  def body(src_r, dst_r, feat_r, ex_r, out_r,
           acc, fbuf, zbuf, src_v, dst_v, exv, sem):
    c = lax.axis_index("c")
    s = lax.axis_index("s")
    w = s * NC + c
    zeros16 = jnp.zeros((LANES,), jnp.float32)

    row0 = s * 640
    ncopies = jnp.where(s == NS - 1, 5, 8)

    def zrow(r, carry):
      for j in range(128 // LANES):
        zbuf[r, pl.ds(j * LANES, LANES)] = zeros16
      return carry
    lax.fori_loop(0, K, zrow, 0)

    def zcp(k, carry):
      pltpu.sync_copy(zbuf, acc.at[pl.ds(row0 + k * K, K), :])
      return carry
    lax.fori_loop(0, ncopies, zcp, 0)
    plsc.subcore_barrier()

    def chunk(i, carry):
      base = w * EW + i * K
      pltpu.sync_copy(src_r.at[pl.ds(base, K)], src_v)
      pltpu.sync_copy(dst_r.at[pl.ds(base, K)], dst_v)
      pltpu.async_copy(feat_r.at[src_v], fbuf, sem).wait()
      for h2 in range(H2):
        pltpu.sync_copy(ex_r.at[pl.ds(h2 * E + base, K)], exv.at[h2])

      def row(rr, rcarry):
        for h2 in range(H2):
          wsp = plsc.load_gather(
              exv, [jnp.full((LANES,), h2, jnp.int32),
                    jnp.full((LANES,), rr, jnp.int32)])
          for q in range(nq):
            col = h2 * 64 + q * LANES if H2 == 2 else q * LANES
            fbuf[rr, pl.ds(col, LANES)] = fbuf[rr, pl.ds(col, LANES)] * wsp
        return rcarry
      lax.fori_loop(0, K, row, 0)
      pltpu.sync_copy(fbuf, acc.at[dst_v], add=True)
      return carry
    lax.fori_loop(0, NCH, chunk, 0)
    plsc.subcore_barrier()

    def ecp(k, carry):
      pltpu.sync_copy(acc.at[pl.ds(row0 + k * K, K), :],
                      out_r.at[c, pl.ds(row0 + k * K, K), :])
      return carry
    lax.fori_loop(0, ncopies, ecp, 0)

  return pl.kernel(
      body,
      out_type=jax.ShapeDtypeStruct((NC, N, 128), jnp.float32),
      mesh=_mesh,
      compiler_params=pltpu.CompilerParams(needs_layout_passes=False),
      scratch_types=[
          pltpu.VMEM_SHARED((N, 128), jnp.float32),
          pltpu.VMEM((K, 128), jnp.float32),
          pltpu.VMEM((K, 128), jnp.float32),
          pltpu.VMEM((K,), jnp.int32),
          pltpu.VMEM((K,), jnp.int32),
          pltpu.VMEM((H2, K), jnp.float32),
          pltpu.SemaphoreType.DMA,
      ],
  )


_edge_scores4 = _make_edge_scores(4)
_edge_scores1 = _make_edge_scores(1)
_aggregate2 = _make_aggregate(2)
_aggregate1 = _make_aggregate(1)


def _t1_body(h_ref, w1_ref, al_ref, ar_ref, featc_ref, el_ref, er_ref):
  feat = jnp.dot(h_ref[...], w1_ref[...], preferred_element_type=jnp.float32)
  for cdx in range(4):
    featc_ref[cdx] = feat[:, cdx * 128:(cdx + 1) * 128]
  fr = feat.reshape(BN, HEADS, HID)
  el_ref[...] = jnp.sum(fr * al_ref[...], axis=-1)
  er_ref[...] = jnp.sum(fr * ar_ref[...], axis=-1)


_t1 = pl.pallas_call(
    _t1_body,
    grid=(GRID,),
    in_specs=[
        pl.BlockSpec((BN, IN_DIM), lambda i: (i, 0)),
        pl.BlockSpec((IN_DIM, HEADS * HID), lambda i: (0, 0)),
        pl.BlockSpec((HEADS, HID), lambda i: (0, 0)),
        pl.BlockSpec((HEADS, HID), lambda i: (0, 0)),
    ],
    out_specs=[
        pl.BlockSpec((4, BN, 128), lambda i: (0, i, 0)),
        pl.BlockSpec((BN, HEADS), lambda i: (i, 0)),
        pl.BlockSpec((BN, HEADS), lambda i: (i, 0)),
    ],
    out_shape=[
        jax.ShapeDtypeStruct((4, N, 128), jnp.float32),
        jax.ShapeDtypeStruct((N, HEADS), jnp.float32),
        jax.ShapeDtypeStruct((N, HEADS), jnp.float32),
    ],
)


BN3 = 200
GRID3 = N // BN3


def _t3_body(accs_ref, s0_ref, s1_ref, b1_ref, w2_ref, al2_ref, ar2_ref,
             featc2_ref, el2_ref, er2_ref):
  acc = jnp.sum(accs_ref[...], axis=1)
  s = jnp.concatenate([jnp.sum(s0_ref[...], axis=0),
                       jnp.sum(s1_ref[...], axis=0)], axis=-1)
  r = 1.0 / s
  parts = []
  for cdx in range(4):
    for hh in range(2):
      hd = cdx * 2 + hh
      seg = acc[cdx][:, hh * 64:(hh + 1) * 64] * r[:, hd][:, None] + b1_ref[hd]
      parts.append(seg)
  h1 = jnp.concatenate(parts, axis=-1)
  h1 = jnp.where(h1 > 0, h1, jnp.exp(jnp.minimum(h1, 0.0)) - 1.0)
  feat2 = jnp.dot(h1, w2_ref[...], preferred_element_type=jnp.float32)
  featc2_ref[...] = feat2
  el2_ref[...] = jnp.sum(feat2 * al2_ref[...], axis=-1)[:, None]
  er2_ref[...] = jnp.sum(feat2 * ar2_ref[...], axis=-1)[:, None]


_t3 = pl.pallas_call(
    _t3_body,
    grid=(GRID3,),
    in_specs=[
        pl.BlockSpec((4, NC, BN3, 128), lambda i: (0, 0, i, 0)),
        pl.BlockSpec((NW, BN3, 4), lambda i: (0, i, 0)),
        pl.BlockSpec((NW, BN3, 4), lambda i: (0, i, 0)),
        pl.BlockSpec((HEADS, HID), lambda i: (0, 0)),
        pl.BlockSpec((HEADS * HID, OUT), lambda i: (0, 0)),
        pl.BlockSpec((1, OUT), lambda i: (0, 0)),
        pl.BlockSpec((1, OUT), lambda i: (0, 0)),
    ],
    out_specs=[
        pl.BlockSpec((BN3, OUT), lambda i: (i, 0)),
        pl.BlockSpec((BN3, 1), lambda i: (i, 0)),
        pl.BlockSpec((BN3, 1), lambda i: (i, 0)),
    ],
    out_shape=[
        jax.ShapeDtypeStruct((N, OUT), jnp.float32),
        jax.ShapeDtypeStruct((N, 1), jnp.float32),
        jax.ShapeDtypeStruct((N, 1), jnp.float32),
    ],
)


def _t5_body(acc2_ref, s2p_ref, b2_ref, out_ref):
  s2 = jnp.sum(s2p_ref[...], axis=0)
  out_ref[...] = (jnp.sum(acc2_ref[...], axis=0) * (1.0 / s2)
                  + b2_ref[...])


_t5 = pl.pallas_call(
    _t5_body,
    grid=(GRID,),
    in_specs=[
        pl.BlockSpec((NC, BN, 128), lambda i: (0, i, 0)),
        pl.BlockSpec((NW, BN, 1), lambda i: (0, i, 0)),
        pl.BlockSpec((1, OUT), lambda i: (0, 0)),
    ],
    out_specs=pl.BlockSpec((BN, OUT), lambda i: (i, 0)),
    out_shape=jax.ShapeDtypeStruct((N, OUT), jnp.float32),
)


def _leaky(x):
  return jnp.maximum(x, 0.2 * x)


def kernel(h, edge_index, W1, attn_l1, attn_r1, b1, W2, attn_l2, attn_r2, b2):
  src = edge_index[0]
  dst = edge_index[1]

  featc1, el1, er1 = _t1(h, W1, attn_l1.reshape(HEADS, HID),
                         attn_r1.reshape(HEADS, HID))

  m1 = _leaky(jnp.max(el1, axis=0) + jnp.max(er1, axis=0))
  m_a = jnp.broadcast_to(m1[:4, None], (4, LANES))
  m_b = jnp.broadcast_to(m1[4:, None], (4, LANES))

  ex_a, s_a = _edge_scores4(src, dst, el1[:, :4].reshape(-1),
                            er1[:, :4].reshape(-1), m_a)
  ex_b, s_b = _edge_scores4(src, dst, el1[:, 4:].reshape(-1),
                            er1[:, 4:].reshape(-1), m_b)

  accs = []
  for cdx in range(4):
    exh = lax.dynamic_slice_in_dim((ex_a if cdx < 2 else ex_b),
                                   (cdx % 2) * 2 * E, 2 * E)
    accs.append(_aggregate2(src, dst, featc1[cdx], exh))
  accs = jnp.stack(accs)

  featc2, el2, er2 = _t3(accs, s_a.reshape(NW, N, 4), s_b.reshape(NW, N, 4),
                         b1.reshape(HEADS, HID), W2,
                         attn_l2.reshape(1, OUT), attn_r2.reshape(1, OUT))

  m2 = _leaky(jnp.max(el2) + jnp.max(er2))
  m2v = jnp.full((1, LANES), m2, jnp.float32)

  ex2, s2p = _edge_scores1(src, dst, el2.reshape(-1), er2.reshape(-1), m2v)
  acc2 = _aggregate1(src, dst, featc2, ex2)

  return _t5(acc2, s2p.reshape(NW, N, 1), b2.reshape(1, OUT))


# speedup vs baseline: 19.2245x; 1.3719x over previous
import functools

import jax
import jax.numpy as jnp
from jax import lax
from jax.experimental import pallas as pl
from jax.experimental.pallas import tpu as pltpu
from jax.experimental.pallas import tpu_sc as plsc

N = 10000
E = 320000
IN_DIM = 128
HID = 64
HEADS = 8
OUT = 128

NC = 2
NS = 16
NW = NC * NS
LANES = 16
EW = E // NW
K = 80
NCH = EW // K
RPT = N // NS

BN = 1000
GRID = N // BN

_mesh = plsc.VectorSubcoreMesh(
    core_axis_name="c", subcore_axis_name="s", num_cores=NC, num_subcores=NS)


def _make_edge_scores(H):
  def body(src_r, dst_r, el_r, er_r, m_r, ex_r, sp_r,
           el_v, er_v, s_v, src_v, dst_v, exb, m_v):
    c = lax.axis_index("c")
    s = lax.axis_index("s")
    w = s * NC + c
    pltpu.sync_copy(el_r, el_v)
    pltpu.sync_copy(er_r, er_v)
    pltpu.sync_copy(m_r, m_v)
    zeros16 = jnp.zeros((LANES,), jnp.float32)

    def zb(i, carry):
      s_v[pl.ds(i * LANES, LANES)] = zeros16
      return carry
    lax.fori_loop(0, (N * H) // LANES, zb, 0)

    mh = [m_v[h, pl.ds(0, LANES)] for h in range(H)]

    def chunk(i, carry):
      base = w * EW + i * K
      pltpu.sync_copy(src_r.at[pl.ds(base, K)], src_v)
      pltpu.sync_copy(dst_r.at[pl.ds(base, K)], dst_v)
      for g in range(K // LANES):
        sv = src_v[pl.ds(g * LANES, LANES)]
        dv = dst_v[pl.ds(g * LANES, LANES)]
        for h in range(H):
          il = plsc.load_gather(el_v, [sv * H + h])
          ir = plsc.load_gather(er_v, [dv * H + h])
          x = il + ir
          x = jnp.maximum(x, 0.2 * x)
          exv = jnp.exp(x - mh[h])
          plsc.addupdate_scatter(s_v, [dv * H + h], exv)
          exb[h, pl.ds(g * LANES, LANES)] = exv
      for h in range(H):
        pltpu.sync_copy(exb.at[h], ex_r.at[pl.ds(h * E + base, K)])
      return carry
    lax.fori_loop(0, NCH, chunk, 0)
    pltpu.sync_copy(s_v, sp_r.at[pl.ds(w * N * H, N * H)])

  return pl.kernel(
      body,
      out_type=(jax.ShapeDtypeStruct((H * E,), jnp.float32),
                jax.ShapeDtypeStruct((NW * N * H,), jnp.float32)),
      mesh=_mesh,
      compiler_params=pltpu.CompilerParams(needs_layout_passes=False),
      scratch_types=[
          pltpu.VMEM((N * H,), jnp.float32),
          pltpu.VMEM((N * H,), jnp.float32),
          pltpu.VMEM((N * H,), jnp.float32),
          pltpu.VMEM((K,), jnp.int32),
          pltpu.VMEM((K,), jnp.int32),
          pltpu.VMEM((H, K), jnp.float32),
          pltpu.VMEM((H, LANES), jnp.float32),
      ],
  )


def _make_aggregate(H2):
  nq = 128 // (LANES * H2)

  def body(src_r, dst_r, feat_r, ex_r, out_r,
           acc, fbuf, src_w, dst_w, dst_v, exw, sem):
    c = lax.axis_index("c")
    s = lax.axis_index("s")
    w = s * NC + c
    zeros16 = jnp.zeros((LANES,), jnp.float32)

    pltpu.sync_copy(src_r.at[pl.ds(w * EW, EW)], src_w)
    pltpu.sync_copy(dst_r.at[pl.ds(w * EW, EW)], dst_w)
    for h2 in range(H2):
      pltpu.sync_copy(ex_r.at[pl.ds(h2 * E + w * EW, EW)],
                      exw.at[pl.ds(h2 * EW, EW)])

    row0 = s * 640
    ncopies = jnp.where(s == NS - 1, 5, 8)

    def zrow(r, carry):
      for j in range(128 // LANES):
        fbuf[r, pl.ds(j * LANES, LANES)] = zeros16
      return carry
    lax.fori_loop(0, K, zrow, 0)

    def zcp(k, carry):
      pltpu.sync_copy(fbuf, acc.at[pl.ds(row0 + k * K, K), :])
      return carry
    lax.fori_loop(0, ncopies, zcp, 0)
    plsc.subcore_barrier()

    def chunk(i, carry):
      pltpu.async_copy(feat_r.at[src_w.at[pl.ds(i * K, K)]], fbuf, sem).wait()
      for g in range(K // LANES):
        dst_v[pl.ds(g * LANES, LANES)] = dst_w[pl.ds(i * K + g * LANES, LANES)]

      def row(rr, rcarry):
        for h2 in range(H2):
          wsp = plsc.load_gather(
              exw, [jnp.full((LANES,), h2 * EW + i * K + rr, jnp.int32)])
          for q in range(nq):
            col = h2 * 64 + q * LANES if H2 == 2 else q * LANES
            fbuf[rr, pl.ds(col, LANES)] = fbuf[rr, pl.ds(col, LANES)] * wsp
        return rcarry
      lax.fori_loop(0, K, row, 0)
      pltpu.sync_copy(fbuf, acc.at[dst_v], add=True)
      return carry
    lax.fori_loop(0, NCH, chunk, 0)
    plsc.subcore_barrier()

    def ecp(k, carry):
      pltpu.sync_copy(acc.at[pl.ds(row0 + k * K, K), :],
                      out_r.at[c, pl.ds(row0 + k * K, K), :])
      return carry
    lax.fori_loop(0, ncopies, ecp, 0)

  return pl.kernel(
      body,
      out_type=jax.ShapeDtypeStruct((NC, N, 128), jnp.float32),
      mesh=_mesh,
      compiler_params=pltpu.CompilerParams(needs_layout_passes=False),
      scratch_types=[
          pltpu.VMEM_SHARED((N, 128), jnp.float32),
          pltpu.VMEM((K, 128), jnp.float32),
          pltpu.VMEM((EW,), jnp.int32),
          pltpu.VMEM((EW,), jnp.int32),
          pltpu.VMEM((K,), jnp.int32),
          pltpu.VMEM((H2 * EW,), jnp.float32),
          pltpu.SemaphoreType.DMA,
      ],
  )


_edge_scores4 = _make_edge_scores(4)
_edge_scores1 = _make_edge_scores(1)
_aggregate2 = _make_aggregate(2)
_aggregate1 = _make_aggregate(1)


def _t1_body(h_ref, w1_ref, al_ref, ar_ref, featc_ref, el_ref, er_ref):
  feat = jnp.dot(h_ref[...], w1_ref[...], preferred_element_type=jnp.float32)
  for cdx in range(4):
    featc_ref[cdx] = feat[:, cdx * 128:(cdx + 1) * 128]
  fr = feat.reshape(BN, HEADS, HID)
  el_ref[...] = jnp.sum(fr * al_ref[...], axis=-1)
  er_ref[...] = jnp.sum(fr * ar_ref[...], axis=-1)


_t1 = pl.pallas_call(
    _t1_body,
    grid=(GRID,),
    in_specs=[
        pl.BlockSpec((BN, IN_DIM), lambda i: (i, 0)),
        pl.BlockSpec((IN_DIM, HEADS * HID), lambda i: (0, 0)),
        pl.BlockSpec((HEADS, HID), lambda i: (0, 0)),
        pl.BlockSpec((HEADS, HID), lambda i: (0, 0)),
    ],
    out_specs=[
        pl.BlockSpec((4, BN, 128), lambda i: (0, i, 0)),
        pl.BlockSpec((BN, HEADS), lambda i: (i, 0)),
        pl.BlockSpec((BN, HEADS), lambda i: (i, 0)),
    ],
    out_shape=[
        jax.ShapeDtypeStruct((4, N, 128), jnp.float32),
        jax.ShapeDtypeStruct((N, HEADS), jnp.float32),
        jax.ShapeDtypeStruct((N, HEADS), jnp.float32),
    ],
)


BN3 = 200
GRID3 = N // BN3


def _t3_body(accs_ref, s0_ref, s1_ref, b1_ref, w2_ref, al2_ref, ar2_ref,
             featc2_ref, el2_ref, er2_ref):
  acc = jnp.sum(accs_ref[...], axis=1)
  s = jnp.concatenate([jnp.sum(s0_ref[...], axis=0),
                       jnp.sum(s1_ref[...], axis=0)], axis=-1)
  r = 1.0 / s
  parts = []
  for cdx in range(4):
    for hh in range(2):
      hd = cdx * 2 + hh
      seg = acc[cdx][:, hh * 64:(hh + 1) * 64] * r[:, hd][:, None] + b1_ref[hd]
      parts.append(seg)
  h1 = jnp.concatenate(parts, axis=-1)
  h1 = jnp.where(h1 > 0, h1, jnp.exp(jnp.minimum(h1, 0.0)) - 1.0)
  feat2 = jnp.dot(h1, w2_ref[...], preferred_element_type=jnp.float32)
  featc2_ref[...] = feat2
  el2_ref[...] = jnp.sum(feat2 * al2_ref[...], axis=-1)[:, None]
  er2_ref[...] = jnp.sum(feat2 * ar2_ref[...], axis=-1)[:, None]


_t3 = pl.pallas_call(
    _t3_body,
    grid=(GRID3,),
    in_specs=[
        pl.BlockSpec((4, NC, BN3, 128), lambda i: (0, 0, i, 0)),
        pl.BlockSpec((NW, BN3, 4), lambda i: (0, i, 0)),
        pl.BlockSpec((NW, BN3, 4), lambda i: (0, i, 0)),
        pl.BlockSpec((HEADS, HID), lambda i: (0, 0)),
        pl.BlockSpec((HEADS * HID, OUT), lambda i: (0, 0)),
        pl.BlockSpec((1, OUT), lambda i: (0, 0)),
        pl.BlockSpec((1, OUT), lambda i: (0, 0)),
    ],
    out_specs=[
        pl.BlockSpec((BN3, OUT), lambda i: (i, 0)),
        pl.BlockSpec((BN3, 1), lambda i: (i, 0)),
        pl.BlockSpec((BN3, 1), lambda i: (i, 0)),
    ],
    out_shape=[
        jax.ShapeDtypeStruct((N, OUT), jnp.float32),
        jax.ShapeDtypeStruct((N, 1), jnp.float32),
        jax.ShapeDtypeStruct((N, 1), jnp.float32),
    ],
)


def _t5_body(acc2_ref, s2p_ref, b2_ref, out_ref):
  s2 = jnp.sum(s2p_ref[...], axis=0)
  out_ref[...] = (jnp.sum(acc2_ref[...], axis=0) * (1.0 / s2)
                  + b2_ref[...])


_t5 = pl.pallas_call(
    _t5_body,
    grid=(GRID,),
    in_specs=[
        pl.BlockSpec((NC, BN, 128), lambda i: (0, i, 0)),
        pl.BlockSpec((NW, BN, 1), lambda i: (0, i, 0)),
        pl.BlockSpec((1, OUT), lambda i: (0, 0)),
    ],
    out_specs=pl.BlockSpec((BN, OUT), lambda i: (i, 0)),
    out_shape=jax.ShapeDtypeStruct((N, OUT), jnp.float32),
)


def _leaky(x):
  return jnp.maximum(x, 0.2 * x)


def kernel(h, edge_index, W1, attn_l1, attn_r1, b1, W2, attn_l2, attn_r2, b2):
  src = edge_index[0]
  dst = edge_index[1]

  featc1, el1, er1 = _t1(h, W1, attn_l1.reshape(HEADS, HID),
                         attn_r1.reshape(HEADS, HID))

  m1 = _leaky(jnp.max(el1, axis=0) + jnp.max(er1, axis=0))
  m_a = jnp.broadcast_to(m1[:4, None], (4, LANES))
  m_b = jnp.broadcast_to(m1[4:, None], (4, LANES))

  ex_a, s_a = _edge_scores4(src, dst, el1[:, :4].reshape(-1),
                            er1[:, :4].reshape(-1), m_a)
  ex_b, s_b = _edge_scores4(src, dst, el1[:, 4:].reshape(-1),
                            er1[:, 4:].reshape(-1), m_b)

  accs = []
  for cdx in range(4):
    exh = lax.dynamic_slice_in_dim((ex_a if cdx < 2 else ex_b),
                                   (cdx % 2) * 2 * E, 2 * E)
    accs.append(_aggregate2(src, dst, featc1[cdx], exh))
  accs = jnp.stack(accs)

  featc2, el2, er2 = _t3(accs, s_a.reshape(NW, N, 4), s_b.reshape(NW, N, 4),
                         b1.reshape(HEADS, HID), W2,
                         attn_l2.reshape(1, OUT), attn_r2.reshape(1, OUT))

  m2 = _leaky(jnp.max(el2) + jnp.max(er2))
  m2v = jnp.full((1, LANES), m2, jnp.float32)

  ex2, s2p = _edge_scores1(src, dst, el2.reshape(-1), er2.reshape(-1), m2v)
  acc2 = _aggregate1(src, dst, featc2, ex2)

  return _t5(acc2, s2p.reshape(NW, N, 1), b2.reshape(1, OUT))
